# baseline (speedup 1.0000x reference)
import jax
import jax.numpy as jnp
from jax.experimental import pallas as pl
from jax.experimental.pallas import tpu as pltpu

B, T, S, D = 32, 8, 2048, 1024


def _fused_kernel(x_hbm, mb_hbm, win_hbm, wout_hbm,
                  scores_ref, alpha_ref, attn_ref,
                  h_scr, c_scr, x_scr, win_scr, wout_scr,
                  mb_buf, mb_sem, chunk_sem, x_sem, win_sem, wout_sem):
    b = pl.program_id(0)

    def mb_copy(i, slot):
        return pltpu.make_async_copy(mb_hbm.at[i], mb_buf.at[slot],
                                     mb_sem.at[slot])

    x_copy = pltpu.make_async_copy(x_hbm, x_scr, x_sem)
    win_copy = pltpu.make_async_copy(win_hbm, win_scr, win_sem)
    wout_copy = pltpu.make_async_copy(wout_hbm, wout_scr, wout_sem)

    slot = jax.lax.rem(b, 2)
    S2 = S // 2

    def chunk0_copy(j):
        return pltpu.make_async_copy(
            mb_hbm.at[0, pl.ds(j * S2, S2), :],
            mb_buf.at[0, pl.ds(j * S2, S2), :],
            chunk_sem.at[j])

    def softmax_store_context(s, row, mb):
        scores_ref[0] = s
        m = jnp.max(s, axis=-1, keepdims=True)
        e = jnp.exp(s - m)
        denom = jnp.sum(e, axis=-1, keepdims=True)
        a = e / denom
        alpha_ref[0] = a
        c_scr[pl.ds(row * T, T), :] = jnp.dot(
            a, mb, preferred_element_type=jnp.float32)

    @pl.when(b == 0)
    def _prologue():
        chunk0_copy(0).start()
        x_copy.start()
        win_copy.start()
        chunk0_copy(1).start()
        mb_copy(1, 1).start()
        wout_copy.start()
        x_copy.wait()
        win_copy.wait()
        h_scr[...] = jax.lax.dot_general(
            x_scr[...], win_scr[...], (((1,), (1,)), ((), ())),
            preferred_element_type=jnp.float32)
        h0 = h_scr[pl.ds(0, T), :]
        chunk0_copy(0).wait()
        s_lo = jax.lax.dot_general(
            h0, mb_buf[0, :S2, :], (((1,), (1,)), ((), ())),
            preferred_element_type=jnp.float32)
        chunk0_copy(1).wait()
        s_hi = jax.lax.dot_general(
            h0, mb_buf[0, S2:, :], (((1,), (1,)), ((), ())),
            preferred_element_type=jnp.float32)
        softmax_store_context(jnp.concatenate([s_lo, s_hi], axis=-1),
                              0, mb_buf[0])

    @pl.when(b > 0)
    def _main():
        mb_copy(b, slot).wait()
        h = h_scr[pl.ds(b * T, T), :]
        mb = mb_buf[slot]
        s = jax.lax.dot_general(h, mb, (((1,), (1,)), ((), ())),
                                preferred_element_type=jnp.float32)
        softmax_store_context(s, b, mb)

    @pl.when(b < B - 2)
    def _prefetch_next():
        mb_copy(b + 2, slot).start()

    @pl.when(b == B - 1)
    def _epilogue():
        wout_copy.wait()
        w_c = wout_scr[:, :D]
        w_x = wout_scr[:, D:]
        out = (jax.lax.dot_general(c_scr[...], w_c, (((1,), (1,)), ((), ())),
                                   preferred_element_type=jnp.float32)
               + jax.lax.dot_general(x_scr[...], w_x, (((1,), (1,)), ((), ())),
                                     preferred_element_type=jnp.float32))
        attn_ref[...] = jnp.tanh(out)


def kernel(input, memory_bank, W_in, W_out):
    x2d = input.reshape(B * T, D)

    scores, alpha, attn2d = pl.pallas_call(
        _fused_kernel,
        grid=(B,),
        in_specs=[
            pl.BlockSpec(memory_space=pl.ANY),
            pl.BlockSpec(memory_space=pl.ANY),
            pl.BlockSpec(memory_space=pl.ANY),
            pl.BlockSpec(memory_space=pl.ANY),
        ],
        out_specs=(
            pl.BlockSpec((1, T, S), lambda b: (b, 0, 0)),
            pl.BlockSpec((1, T, S), lambda b: (b, 0, 0)),
            pl.BlockSpec((B * T, D), lambda b: (0, 0)),
        ),
        out_shape=(
            jax.ShapeDtypeStruct((B, T, S), jnp.float32),
            jax.ShapeDtypeStruct((B, T, S), jnp.float32),
            jax.ShapeDtypeStruct((B * T, D), jnp.float32),
        ),
        scratch_shapes=[
            pltpu.VMEM((B * T, D), jnp.float32),
            pltpu.VMEM((B * T, D), jnp.float32),
            pltpu.VMEM((B * T, D), jnp.float32),
            pltpu.VMEM((D, D), jnp.float32),
            pltpu.VMEM((D, 2 * D), jnp.float32),
            pltpu.VMEM((2, S, D), jnp.float32),
            pltpu.SemaphoreType.DMA((2,)),
            pltpu.SemaphoreType.DMA((2,)),
            pltpu.SemaphoreType.DMA,
            pltpu.SemaphoreType.DMA,
            pltpu.SemaphoreType.DMA,
        ],
    )(x2d, memory_bank, W_in, W_out)

    return (attn2d.reshape(B, T, D), alpha, scores)


# speedup vs baseline: 1.1106x; 1.1106x over previous
import jax
import jax.numpy as jnp
from jax.experimental import pallas as pl
from jax.experimental.pallas import tpu as pltpu

B, T, S, D = 32, 8, 2048, 1024


def _fused_kernel(x_hbm, mb_hbm, win_hbm, wout_hbm,
                  scores_ref, alpha_ref, attn_ref,
                  h_scr, c_scr, x_scr, win_scr, wout_scr,
                  mb_buf, mb_sem, x_sem, win_sem, wout_sem):
    b = pl.program_id(0)

    def mb_copy(i, slot):
        return pltpu.make_async_copy(mb_hbm.at[i], mb_buf.at[slot],
                                     mb_sem.at[slot])

    x_copy = pltpu.make_async_copy(x_hbm, x_scr, x_sem)
    win_copy = pltpu.make_async_copy(win_hbm, win_scr, win_sem)
    wout_copy = pltpu.make_async_copy(wout_hbm, wout_scr, wout_sem)

    slot = jax.lax.rem(b, 2)

    @pl.when(b == 0)
    def _prologue():
        mb_copy(0, 0).start()
        x_copy.start()
        win_copy.start()
        mb_copy(1, 1).start()
        wout_copy.start()
        x_copy.wait()
        win_copy.wait()
        h_scr[...] = jax.lax.dot_general(
            x_scr[...], win_scr[...], (((1,), (1,)), ((), ())),
            preferred_element_type=jnp.float32)

    mb_copy(b, slot).wait()

    h = h_scr[pl.ds(b * T, T), :]
    mb = mb_buf[slot]
    s = jax.lax.dot_general(h, mb, (((1,), (1,)), ((), ())),
                            preferred_element_type=jnp.float32)
    scores_ref[0] = s
    m = jnp.max(s, axis=-1, keepdims=True)
    e = jnp.exp(s - m)
    denom = jnp.sum(e, axis=-1, keepdims=True)
    a = e / denom
    alpha_ref[0] = a
    c_scr[pl.ds(b * T, T), :] = jnp.dot(a, mb,
                                        preferred_element_type=jnp.float32)

    @pl.when(b < B - 2)
    def _prefetch_next():
        mb_copy(b + 2, slot).start()

    @pl.when(b == B - 1)
    def _epilogue():
        wout_copy.wait()
        w_c = wout_scr[:, :D]
        w_x = wout_scr[:, D:]
        out = (jax.lax.dot_general(c_scr[...], w_c, (((1,), (1,)), ((), ())),
                                   preferred_element_type=jnp.float32)
               + jax.lax.dot_general(x_scr[...], w_x, (((1,), (1,)), ((), ())),
                                     preferred_element_type=jnp.float32))
        attn_ref[...] = jnp.tanh(out)


def kernel(input, memory_bank, W_in, W_out):
    x2d = input.reshape(B * T, D)

    scores, alpha, attn2d = pl.pallas_call(
        _fused_kernel,
        grid=(B,),
        in_specs=[
            pl.BlockSpec(memory_space=pl.ANY),
            pl.BlockSpec(memory_space=pl.ANY),
            pl.BlockSpec(memory_space=pl.ANY),
            pl.BlockSpec(memory_space=pl.ANY),
        ],
        out_specs=(
            pl.BlockSpec((1, T, S), lambda b: (b, 0, 0)),
            pl.BlockSpec((1, T, S), lambda b: (b, 0, 0)),
            pl.BlockSpec((B * T, D), lambda b: (0, 0)),
        ),
        out_shape=(
            jax.ShapeDtypeStruct((B, T, S), jnp.float32),
            jax.ShapeDtypeStruct((B, T, S), jnp.float32),
            jax.ShapeDtypeStruct((B * T, D), jnp.float32),
        ),
        scratch_shapes=[
            pltpu.VMEM((B * T, D), jnp.float32),
            pltpu.VMEM((B * T, D), jnp.float32),
            pltpu.VMEM((B * T, D), jnp.float32),
            pltpu.VMEM((D, D), jnp.float32),
            pltpu.VMEM((D, 2 * D), jnp.float32),
            pltpu.VMEM((2, S, D), jnp.float32),
            pltpu.SemaphoreType.DMA((2,)),
            pltpu.SemaphoreType.DMA,
            pltpu.SemaphoreType.DMA,
            pltpu.SemaphoreType.DMA,
        ],
    )(x2d, memory_bank, W_in, W_out)

    return (attn2d.reshape(B, T, D), alpha, scores)
